# baseline (speedup 1.0000x reference)
import functools

import jax
import jax.numpy as jnp
from jax import lax
from jax.experimental import pallas as pl
from jax.experimental.pallas import tpu as pltpu
from jax.experimental.pallas import tpu_sc as plsc

_NODE_TYPES = ['campaign', 'platform', 'channel', 'creative', 'keywords',
               'search_tag', 'advertiser', 'network', 'template', 'region',
               'currency']
_RELS = [('campaign', 'hosted_on', 'platform'),
         ('platform', 'rev_hosted_on', 'campaign'),
         ('campaign', 'uses', 'channel'),
         ('channel', 'rev_uses', 'campaign'),
         ('platform', 'supports', 'channel'),
         ('campaign', 'uses', 'creative'),
         ('creative', 'rev_uses', 'campaign'),
         ('creative', 'designed_with', 'template'),
         ('campaign', 'associated_with', 'keywords'),
         ('keywords', 'rev_associated_with', 'campaign'),
         ('campaign', 'managed_by', 'network'),
         ('platform', 'optimized_for', 'keywords'),
         ('campaign', 'belongs_to', 'advertiser'),
         ('campaign', 'targeted_with', 'search_tag'),
         ('search_tag', 'rev_targeted_with', 'campaign'),
         ('campaign', 'targeted_in', 'region'),
         ('campaign', 'uses', 'currency')]
_T = len(_NODE_TYPES)
_R = len(_RELS)
_SRC = [_NODE_TYPES.index(r[0]) for r in _RELS]
_DST = [_NODE_TYPES.index(r[2]) for r in _RELS]
_N = 10000
_D = 128
_E = 160000
_L = 3

_KT = [sum(1 for d in _DST if d == t) for t in range(_T)]

_NC = 2
_NS = 16
_H = _D // _NC
_RPT = _N // _NS
_AR = 640
_DUMP = 632
_SEG = _E // _NS
_SROW = _SEG // 16
_K = 128
_CAP = 10240
_TRASH = _CAP - 16
_CW = 16

_CORE_RELS = [list(range(9)), list(range(9, _R))]


def _fill_vmem_2d(ref, rows, cols, value, dtype=jnp.float32):
  vec = jnp.full((16,), value, dtype=dtype)
  per_row = cols // 16

  def body(i, _):
    r = i // per_row
    c = (i % per_row) * 16
    ref[r, pl.ds(c, 16)] = vec
    return 0

  lax.fori_loop(0, rows * per_row, body, 0)


def _sc_partition_body(e0_ref, e1_ref, src_out, dstl_out, cnt_out, cn_out,
                       e0seg_v, e1seg_v, srcbuf, dstlbuf, cntv_buf,
                       ones_v, zero16_v, cntacc):
  cid = lax.axis_index("c")
  sid = lax.axis_index("s")
  lo = sid * _RPT
  _fill_vmem_2d(ones_v, _K, _CW, 1.0)
  _fill_vmem_2d(zero16_v, 160, _CW, 0.0)
  lane = lax.iota(jnp.int32, 16)
  base = sid * _AR
  dumpvec = jnp.full((16,), _DUMP, dtype=jnp.int32) + base
  zerovec = jnp.zeros((16,), dtype=jnp.int32)

  for core in range(_NC):
    @pl.when(cid == core)
    def _(core=core):
      for r in _CORE_RELS[core]:
        for z in range(4):
          pltpu.sync_copy(zero16_v, cntacc.at[pl.ds(base + z * 160, 160)])
        def seg_body(seg, cnt_vec):
          pltpu.sync_copy(e0_ref.at[r, seg], e0seg_v)
          pltpu.sync_copy(e1_ref.at[r, seg], e1seg_v)

          def scan(i, off):
            dv = e1seg_v[i]
            sv = e0seg_v[i]
            d = dv - lo
            hit = (jnp.maximum(0, jnp.minimum(1, d + 1))
                   * jnp.maximum(0, jnp.minimum(1, _RPT - d)))
            rank = plsc.cumsum(hit) - hit
            pos = hit * (off + rank) + (1 - hit) * _TRASH
            plsc.store_scatter(dstlbuf, [pos // _K, pos % _K], d + base)
            plsc.store_scatter(srcbuf, [pos // _K, pos % _K], sv)
            return off + jnp.sum(hit)

          cnt = lax.fori_loop(0, _SROW, scan, jnp.int32(0))
          for k in range(8):
            pp = cnt + k * 16 + lane
            plsc.store_scatter(dstlbuf, [pp // _K, pp % _K], dumpvec)
            plsc.store_scatter(srcbuf, [pp // _K, pp % _K], zerovec)
          pltpu.sync_copy(srcbuf, src_out.at[r, sid, seg])
          pltpu.sync_copy(dstlbuf, dstl_out.at[r, sid, seg])

          nch = (cnt + _K - 1) // _K

          def chunk(j, _):
            @pl.when(j < nch)
            def _():
              pltpu.sync_copy(ones_v, cntacc.at[dstlbuf.at[j]], add=True)
            return 0

          lax.fori_loop(0, _CAP // _K, chunk, 0)
          hit = 1 - jnp.minimum(jnp.abs(lane - seg), 1)
          return cnt_vec + hit * cnt

        cnt_vec = lax.fori_loop(0, _NS, seg_body,
                                jnp.zeros((16,), jnp.int32))
        cntv_buf[r, :] = cnt_vec
        pltpu.sync_copy(cntacc.at[pl.ds(base, _RPT)],
                        cn_out.at[r, pl.ds(sid * _RPT, _RPT)])
      pltpu.sync_copy(cntv_buf, cnt_out.at[:, sid])


_sc_partition = pl.kernel(
    _sc_partition_body,
    out_type=(
        jax.ShapeDtypeStruct((_R, _NS, _NS, _CAP // _K, _K), jnp.int32),
        jax.ShapeDtypeStruct((_R, _NS, _NS, _CAP // _K, _K), jnp.int32),
        jax.ShapeDtypeStruct((_R, _NS, _NS), jnp.int32),
        jax.ShapeDtypeStruct((_R, _N, _CW), jnp.float32),
    ),
    mesh=plsc.VectorSubcoreMesh(core_axis_name="c", subcore_axis_name="s",
                                num_cores=_NC, num_subcores=_NS),
    scratch_types=[
        pltpu.VMEM((_SROW, 16), jnp.int32),
        pltpu.VMEM((_SROW, 16), jnp.int32),
        pltpu.VMEM((_CAP // _K, _K), jnp.int32),
        pltpu.VMEM((_CAP // _K, _K), jnp.int32),
        pltpu.VMEM((_R, 16), jnp.int32),
        pltpu.VMEM((_K, _CW), jnp.float32),
        pltpu.VMEM((160, _CW), jnp.float32),
        pltpu.VMEM_SHARED((_NS * _AR, _CW), jnp.float32),
    ],
    compiler_params=pltpu.CompilerParams(use_tc_tiling_on_sc=False,
                                         needs_layout_passes=False),
)


def _sc_sums_body(*refs):
  xlo = refs[:_T]
  xhi = refs[_T:2 * _T]
  src_ref, dstl_ref, cnt_ref, out_ref = refs[2 * _T:2 * _T + 4]
  (srcidx_v, dstidx_v, idx128, buf, zero_v, cntv, acc,
   sem) = refs[2 * _T + 4:]
  cid = lax.axis_index("c")
  sid = lax.axis_index("s")
  lane = lax.iota(jnp.int32, 16)
  _fill_vmem_2d(zero_v, 160, _H, 0.0)
  base = sid * _AR
  pltpu.sync_copy(cnt_ref.at[:, sid], cntv)
  for core in range(_NC):
    @pl.when(cid == core)
    def _(core=core):
      halves = xlo if core == 0 else xhi
      for r in range(_R):
        half = halves[_SRC[r]]
        for z in range(4):
          pltpu.sync_copy(zero_v, acc.at[pl.ds(base + z * 160, 160)])
        cnt_row = cntv[r]

        def seg_body(seg, _):
          pltpu.sync_copy(src_ref.at[r, sid, seg], srcidx_v)
          pltpu.sync_copy(dstl_ref.at[r, sid, seg], dstidx_v)
          hit = 1 - jnp.minimum(jnp.abs(lane - seg), 1)
          cnt = jnp.sum(hit * cnt_row)
          nch = (cnt + _K - 1) // _K

          def chunk(j, _):
            @pl.when(j < nch)
            def _():
              for k in range(8):
                idx128[pl.ds(k * 16, 16)] = srcidx_v[j, pl.ds(k * 16, 16)]
              pltpu.async_copy(half.at[idx128], buf, sem).wait()
              pltpu.sync_copy(buf, acc.at[dstidx_v.at[j]], add=True)
            return 0

          lax.fori_loop(0, _CAP // _K, chunk, 0)
          return 0

        lax.fori_loop(0, _NS, seg_body, 0)
        pltpu.sync_copy(acc.at[pl.ds(base, _RPT)],
                        out_ref.at[r, core, pl.ds(sid * _RPT, _RPT)])


_sc_sums = pl.kernel(
    _sc_sums_body,
    out_type=jax.ShapeDtypeStruct((_R, _NC, _N, _H), jnp.float32),
    mesh=plsc.VectorSubcoreMesh(core_axis_name="c", subcore_axis_name="s",
                                num_cores=_NC, num_subcores=_NS),
    scratch_types=[
        pltpu.VMEM((_CAP // _K, _K), jnp.int32),
        pltpu.VMEM((_CAP // _K, _K), jnp.int32),
        pltpu.VMEM((_K,), jnp.int32),
        pltpu.VMEM((_K, _H), jnp.float32),
        pltpu.VMEM((160, _H), jnp.float32),
        pltpu.VMEM((_R, 16), jnp.int32),
        pltpu.VMEM_SHARED((_NS * _AR, _H), jnp.float32),
        pltpu.SemaphoreType.DMA,
    ],
    compiler_params=pltpu.CompilerParams(use_tc_tiling_on_sc=False,
                                         needs_layout_passes=False),
)


_B = 400
_GRID = _N // _B


def _tc_layer_body(last, *refs):
  s_ref, c_ref = refs[0], refs[1]
  xs = refs[2:2 + 2 * _T]
  wl_ref, wr_ref, bias_ref, g_ref, b_ref = refs[2 + 2 * _T:7 + 2 * _T]
  pos = 7 + 2 * _T
  if last:
    fcw_ref, fcb_ref = refs[pos], refs[pos + 1]
    pos += 2
  outs = refs[pos:pos + 2 * _T]
  if last:
    p_ref = refs[pos + 2 * _T]

  h = [None] * _T
  for t in range(_T):
    xt = jnp.concatenate([xs[2 * t][...], xs[2 * t + 1][...]], axis=1)
    h[t] = (jnp.dot(xt, wr_ref[t],
                    preferred_element_type=jnp.float32,
                    precision=lax.Precision.HIGHEST)
            + bias_ref[t][None, :])
  for r in range(_R):
    cnt = c_ref[r]
    s_r = jnp.concatenate([s_ref[r, 0], s_ref[r, 1]], axis=1)
    mean = s_r / jnp.maximum(cnt, 1.0)
    h[_DST[r]] = h[_DST[r]] + jnp.dot(
        mean, wl_ref[r], preferred_element_type=jnp.float32,
        precision=lax.Precision.HIGHEST)

  acc = None
  gvec = g_ref[...][None, :]
  bvec = b_ref[...][None, :]
  for t in range(_T):
    v = h[t]
    v = jnp.where(v >= 0, v, 0.01 * v)
    mu = jnp.mean(v, axis=1, keepdims=True)
    d = v - mu
    var = jnp.mean(d * d, axis=1, keepdims=True)
    y = d * lax.rsqrt(var + 1e-5) * gvec + bvec
    outs[2 * t][...] = y[:, :_H]
    outs[2 * t + 1][...] = y[:, _H:]
    if last:
      part = jnp.dot(y, fcw_ref[t], preferred_element_type=jnp.float32,
                     precision=lax.Precision.HIGHEST)
      acc = part if acc is None else acc + part
  if last:
    p_ref[...] = jax.nn.sigmoid(acc + fcb_ref[0])[:, None]


def _tc_layer(last, s, c, xs, wl, wr, bias, g, b, fcw=None, fcb=None):
  full = lambda *shape: pl.BlockSpec(shape, lambda i: (0,) * len(shape))
  in_specs = [
      pl.BlockSpec((_R, _NC, _B, _H), lambda i: (0, 0, i, 0)),
      pl.BlockSpec((_R, _B, 1), lambda i: (0, i, 0)),
  ]
  in_specs += [pl.BlockSpec((_B, _H), lambda i: (i, 0))
               for _ in range(2 * _T)]
  in_specs += [full(_R, _D, _D), full(_T, _D, _D), full(_T, _D),
               full(_D,), full(_D,)]
  args = [s, c, *xs, wl, wr, bias, g, b]
  if last:
    in_specs += [full(_T, _D), full(1,)]
    args += [fcw, fcb]
  out_specs = [pl.BlockSpec((_B, _H), lambda i: (i, 0))
               for _ in range(2 * _T)]
  out_shapes = [jax.ShapeDtypeStruct((_N, _H), jnp.float32)
                for _ in range(2 * _T)]
  if last:
    out_specs.append(pl.BlockSpec((_B, 1), lambda i: (i, 0)))
    out_shapes.append(jax.ShapeDtypeStruct((_N, 1), jnp.float32))
  return pl.pallas_call(
      functools.partial(_tc_layer_body, last),
      grid=(_GRID,),
      in_specs=in_specs,
      out_specs=out_specs,
      out_shape=out_shapes,
  )(*args)


def kernel(x_campaign, x_platform, x_channel, x_creative, x_keywords,
           x_search_tag, x_advertiser, x_network, x_template, x_region,
           x_currency, ei_campaign_hosted_on_platform,
           ei_platform_rev_hosted_on_campaign, ei_campaign_uses_channel,
           ei_channel_rev_uses_campaign, ei_platform_supports_channel,
           ei_campaign_uses_creative, ei_creative_rev_uses_campaign,
           ei_creative_designed_with_template,
           ei_campaign_associated_with_keywords,
           ei_keywords_rev_associated_with_campaign,
           ei_campaign_managed_by_network, ei_platform_optimized_for_keywords,
           ei_campaign_belongs_to_advertiser,
           ei_campaign_targeted_with_search_tag,
           ei_search_tag_rev_targeted_with_campaign,
           ei_campaign_targeted_in_region, ei_campaign_uses_currency,
           W_l, b_l, W_r, ln_g, ln_b, fc_w, fc_b):
  xs_full = [x_campaign, x_platform, x_channel, x_creative, x_keywords,
             x_search_tag, x_advertiser, x_network, x_template, x_region,
             x_currency]
  eis = [ei_campaign_hosted_on_platform, ei_platform_rev_hosted_on_campaign,
         ei_campaign_uses_channel, ei_channel_rev_uses_campaign,
         ei_platform_supports_channel, ei_campaign_uses_creative,
         ei_creative_rev_uses_campaign, ei_creative_designed_with_template,
         ei_campaign_associated_with_keywords,
         ei_keywords_rev_associated_with_campaign,
         ei_campaign_managed_by_network, ei_platform_optimized_for_keywords,
         ei_campaign_belongs_to_advertiser,
         ei_campaign_targeted_with_search_tag,
         ei_search_tag_rev_targeted_with_campaign,
         ei_campaign_targeted_in_region, ei_campaign_uses_currency]

  xs = []
  for x in xs_full:
    xs.extend([x[:, :_H], x[:, _H:]])

  e0 = jnp.stack([e[0] for e in eis]).reshape(_R, _NS, _SROW, 16)
  e1 = jnp.stack([e[1] for e in eis]).reshape(_R, _NS, _SROW, 16)

  inv_kt = jnp.array([1.0 / _KT[t] for t in range(_T)], dtype=jnp.float32)
  scale_r = inv_kt[jnp.array(_DST)]
  wl_all = W_l * scale_r[None, :, None, None]
  dst_onehot = jnp.zeros((_T, _R), jnp.float32).at[
      jnp.array(_DST), jnp.arange(_R)].set(1.0)
  wr_all = jnp.einsum('tr,lrde->ltde', dst_onehot, W_r) \
      * inv_kt[None, :, None, None]
  bias_all = jnp.einsum('tr,lrd->ltd', dst_onehot, b_l) \
      * inv_kt[None, :, None]
  fcw = fc_w.reshape(_T, _D)

  src_l, dstl_l, ecnt, cn = _sc_partition(e0, e1)
  cnts = cn[:, :, :1]

  for l in range(_L):
    xfull = [jnp.concatenate([xs[2 * t], xs[2 * t + 1]], axis=1)
             for t in range(_T)]
    s_full = jnp.stack([
        jax.ops.segment_sum(xfull[_SRC[r]][eis[r][0]], eis[r][1],
                            num_segments=_N) for r in range(_R)])
    s = s_full.reshape(_R, _N, _NC, _H).transpose(0, 2, 1, 3)
    last = l == _L - 1
    res = _tc_layer(last, s, cnts, xs, wl_all[l], wr_all[l], bias_all[l],
                    ln_g[l], ln_b[l], fcw if last else None,
                    fc_b if last else None)
    xs = list(res[:2 * _T])
    if last:
      p = res[2 * _T]
  return p[:, 0]
